# baseline (speedup 1.0000x reference)
import functools

import jax
import jax.numpy as jnp
import numpy as np
from jax import lax
from jax.experimental import pallas as pl
from jax.experimental.pallas import tpu as pltpu
from jax.experimental.pallas import tpu_sc as plsc

_TOPK_FRAC = 0.2
_ROWS_PER_BLOCK = 48
_N_SC_ROWS = 32
_NC, _NS, _LANES = 2, 16, 16
_NW = _NC * _NS
_UNROLL = 8


def _flip_topk_kernel(x_ref, o_ref, pfx_ref, *, k, rows, length, idx_bits):
    x = x_ref[...]
    b = jax.lax.bitcast_convert_type(x, jnp.int32)
    s = jnp.where(b < 0, b ^ jnp.int32(0x7FFFFFFF), b)

    kk = jnp.int32(k)

    cnt_nonneg = jnp.sum((s >= 0).astype(jnp.int32), axis=1, keepdims=True)
    base = jnp.where(cnt_nonneg >= kk, jnp.int32(0), jnp.int32(-(2**31)))

    def value_step(i, prefix):
        bit = jnp.int32(1) << (30 - i)
        cand = base | prefix | bit
        cnt = jnp.sum((s >= cand).astype(jnp.int32), axis=1, keepdims=True)
        return jnp.where(cnt >= kk, prefix | bit, prefix)

    prefix = jax.lax.fori_loop(
        0, 31, value_step, jnp.zeros((rows, 1), jnp.int32)
    )
    t = base | prefix

    gt = s > t
    eq = s == t
    cnt_gt = jnp.sum(gt.astype(jnp.int32), axis=1, keepdims=True)
    cnt_eq = jnp.sum(eq.astype(jnp.int32), axis=1, keepdims=True)
    need = kk - cnt_gt

    ridx = jnp.int32(length - 1) - jax.lax.broadcasted_iota(
        jnp.int32, (rows, length), 1
    )
    keys = jnp.where(eq, ridx, jnp.int32(-1))

    pfx_ref[...] = jnp.zeros((rows, 1), jnp.int32)

    @pl.when(jnp.any(cnt_eq != need))
    def _tie_descent():
        def idx_step(i, pfx):
            bit = jnp.int32(1) << (idx_bits - 1 - i)
            cand = pfx | bit
            cnt = jnp.sum((keys >= cand).astype(jnp.int32), axis=1,
                          keepdims=True)
            return jnp.where(cnt >= need, pfx | bit, pfx)

        pfx_ref[...] = jax.lax.fori_loop(
            0, idx_bits, idx_step, jnp.zeros((rows, 1), jnp.int32)
        )

    flip = gt | (eq & (keys >= pfx_ref[...]))
    o_ref[...] = jnp.where(flip, 1.0 - x, x)


def _sc_body(x_hbm, o_hbm, row_v, hist_v, *, k, n_rows, length):
    nvec = length // _LANES
    wid = lax.axis_index("s") * _NC + lax.axis_index("c")
    rows_per_w = n_rows // _NW
    lane = lax.iota(jnp.int32, _LANES)
    lane257 = lane * 257
    ones = jnp.full((_LANES,), 1, jnp.int32)
    kk = jnp.int32(k)

    def key_of(v):
        b = lax.bitcast_convert_type(v, jnp.int32)
        return jnp.where(b < 0, b ^ jnp.int32(0x7FFFFFFF), b)

    def do_row(r, _):
        row = wid * rows_per_w + r
        pltpu.sync_copy(x_hbm.at[row], row_v)

        prefix = jnp.int32(0)
        gt_acc = jnp.int32(0)
        eq_cnt = jnp.int32(0)

        for p in range(4):
            shift = 24 - 8 * p

            def zero_step(j, _c):
                for u in range(_UNROLL):
                    hist_v[pl.ds((j * _UNROLL + u) * _LANES, _LANES)] = (
                        jnp.zeros((_LANES,), jnp.int32)
                    )
                return _c

            lax.fori_loop(0, 264 // _UNROLL, zero_step, jnp.int32(0))

            if p == 0:
                def hist_step(i, _c):
                    for u in range(_UNROLL):
                        s = key_of(
                            row_v[pl.ds((i * _UNROLL + u) * _LANES,
                                        _LANES)]
                        )
                        binv = (s >> 24) + 128
                        plsc.addupdate_scatter(
                            hist_v, [lane257 + binv], ones
                        )
                    return _c
            else:
                himask = jnp.int32(-(1 << (shift + 8)))
                pref_hi = prefix & himask

                def hist_step(i, _c, _himask=himask, _pref=pref_hi,
                              _shift=shift):
                    for u in range(_UNROLL):
                        s = key_of(
                            row_v[pl.ds((i * _UNROLL + u) * _LANES,
                                        _LANES)]
                        )
                        sel = (s & _himask) == _pref
                        binv = (s >> _shift) & 255
                        plsc.addupdate_scatter(
                            hist_v, [lane257 + binv], ones, mask=sel
                        )
                    return _c

            lax.fori_loop(0, nvec // _UNROLL, hist_step, jnp.int32(0))

            remaining = kk - gt_acc
            carry = jnp.int32(0)
            found = jnp.int32(0)
            chosen = jnp.int32(0)
            gt_above = jnp.int32(0)
            hb_ch = jnp.int32(0)
            for g in range(15, -1, -1):
                gv = hist_v[pl.ds(0 * 257 + g * _LANES, _LANES)]
                for l in range(1, _LANES):
                    gv = gv + hist_v[pl.ds(l * 257 + g * _LANES, _LANES)]
                rv = lax.rev(gv, (0,))
                cs = plsc.cumsum(rv)
                tot = jnp.sum(gv)
                hit = (found == 0) & ((carry + tot) >= remaining)
                crossed = (carry + cs) >= remaining
                istar = jnp.min(
                    jnp.where(crossed, lane, jnp.int32(_LANES))
                )
                hb = jnp.sum(jnp.where(lane == istar, rv, 0))
                cs_at = jnp.sum(jnp.where(lane == istar, cs, 0))
                bstar = g * _LANES + (_LANES - 1) - istar
                chosen = jnp.where(hit, bstar, chosen)
                gt_above = jnp.where(hit, carry + cs_at - hb, gt_above)
                hb_ch = jnp.where(hit, hb, hb_ch)
                found = jnp.where(hit, jnp.int32(1), found)
                carry = jnp.where(found == 0, carry + tot, carry)

            byte_bits = (chosen ^ 128) if p == 0 else chosen
            prefix = prefix | (byte_bits << shift)
            gt_acc = gt_acc + gt_above
            eq_cnt = hb_ch

        t = prefix
        need = kk - gt_acc

        @pl.when(eq_cnt == need)
        def _fast():
            def flip_step(i, _c):
                for u in range(_UNROLL):
                    sl = pl.ds((i * _UNROLL + u) * _LANES, _LANES)
                    v = row_v[sl]
                    s = key_of(v)
                    row_v[sl] = jnp.where(s >= t, 1.0 - v, v)
                return _c

            lax.fori_loop(0, nvec // _UNROLL, flip_step, jnp.int32(0))

        @pl.when(eq_cnt != need)
        def _tie():
            def flip_step(i, seen):
                v = row_v[pl.ds(i * _LANES, _LANES)]
                s = key_of(v)
                gtm = s > t
                eqm = s == t
                eqi = eqm.astype(jnp.int32)
                csum = plsc.cumsum(eqi)
                flip = gtm | (eqm & ((csum + seen) <= need))
                row_v[pl.ds(i * _LANES, _LANES)] = jnp.where(
                    flip, 1.0 - v, v
                )
                return seen + jnp.sum(eqi)

            lax.fori_loop(0, nvec, flip_step, jnp.int32(0))

        pltpu.sync_copy(row_v, o_hbm.at[row])
        return _

    lax.fori_loop(0, rows_per_w, do_row, jnp.int32(0))


def _sc_call(flat, k):
    B, L = flat.shape
    mesh = plsc.VectorSubcoreMesh(
        core_axis_name="c", subcore_axis_name="s"
    )
    body = functools.partial(_sc_body, k=k, n_rows=B, length=L)
    fn = pl.kernel(
        body,
        mesh=mesh,
        out_type=jax.ShapeDtypeStruct((B, L), jnp.float32),
        scratch_types=[
            pltpu.VMEM((L,), jnp.float32),
            pltpu.VMEM((4224,), jnp.int32),
        ],
        compiler_params=pltpu.CompilerParams(needs_layout_passes=False),
    )
    return fn(flat)


def kernel(Attention_map):
    B, C, H, W = Attention_map.shape
    L = C * H * W
    k = int(np.clip(int(L * _TOPK_FRAC), 1, C))
    idx_bits = max(int(L - 1).bit_length(), 1)
    flat = Attention_map.reshape(B, L)

    n_sc = _N_SC_ROWS
    out_sc = _sc_call(flat[:n_sc], k)

    rows = _ROWS_PER_BLOCK
    n_tc = B - n_sc
    out_tc = pl.pallas_call(
        functools.partial(
            _flip_topk_kernel, k=k, rows=rows, length=L, idx_bits=idx_bits
        ),
        grid=(n_tc // rows,),
        in_specs=[pl.BlockSpec((rows, L), lambda i: (i, 0))],
        out_specs=pl.BlockSpec((rows, L), lambda i: (i, 0)),
        out_shape=jax.ShapeDtypeStruct((n_tc, L), jnp.float32),
        scratch_shapes=[pltpu.VMEM((rows, 1), jnp.int32)],
        compiler_params=pltpu.CompilerParams(
            dimension_semantics=("parallel",),
        ),
    )(flat[n_sc:])

    out = jnp.concatenate([out_sc, out_tc], axis=0)
    return out.reshape(B, C, H, W)


# speedup vs baseline: 1.0010x; 1.0010x over previous
import functools

import jax
import jax.numpy as jnp
import numpy as np
from jax import lax
from jax.experimental import pallas as pl
from jax.experimental.pallas import tpu as pltpu
from jax.experimental.pallas import tpu_sc as plsc

_TOPK_FRAC = 0.2
_ROWS_PER_BLOCK = 48
_N_SC_ROWS = 32
_NC, _NS, _LANES = 2, 16, 16
_NW = _NC * _NS
_UNROLL = 8


def _flip_topk_kernel(x_ref, o_ref, pfx_ref, *, k, rows, length, idx_bits):
    x = x_ref[...]
    b = jax.lax.bitcast_convert_type(x, jnp.int32)
    s = jnp.where(b < 0, b ^ jnp.int32(0x7FFFFFFF), b)

    kk = jnp.int32(k)

    cnt_nonneg = jnp.sum((s >= 0).astype(jnp.int32), axis=1, keepdims=True)
    base = jnp.where(cnt_nonneg >= kk, jnp.int32(0), jnp.int32(-(2**31)))

    def value_step(i, prefix):
        bit = jnp.int32(1) << (30 - i)
        cand = base | prefix | bit
        cnt = jnp.sum((s >= cand).astype(jnp.int32), axis=1, keepdims=True)
        return jnp.where(cnt >= kk, prefix | bit, prefix)

    prefix = jax.lax.fori_loop(
        0, 31, value_step, jnp.zeros((rows, 1), jnp.int32)
    )
    t = base | prefix

    gt = s > t
    eq = s == t
    cnt_gt = jnp.sum(gt.astype(jnp.int32), axis=1, keepdims=True)
    cnt_eq = jnp.sum(eq.astype(jnp.int32), axis=1, keepdims=True)
    need = kk - cnt_gt

    ridx = jnp.int32(length - 1) - jax.lax.broadcasted_iota(
        jnp.int32, (rows, length), 1
    )
    keys = jnp.where(eq, ridx, jnp.int32(-1))

    pfx_ref[...] = jnp.zeros((rows, 1), jnp.int32)

    @pl.when(jnp.any(cnt_eq != need))
    def _tie_descent():
        def idx_step(i, pfx):
            bit = jnp.int32(1) << (idx_bits - 1 - i)
            cand = pfx | bit
            cnt = jnp.sum((keys >= cand).astype(jnp.int32), axis=1,
                          keepdims=True)
            return jnp.where(cnt >= need, pfx | bit, pfx)

        pfx_ref[...] = jax.lax.fori_loop(
            0, idx_bits, idx_step, jnp.zeros((rows, 1), jnp.int32)
        )

    flip = gt | (eq & (keys >= pfx_ref[...]))
    o_ref[...] = jnp.where(flip, 1.0 - x, x)


def _sc_body(x_hbm, o_hbm, row_v, hist_v, *, k, n_rows, length):
    nvec = length // _LANES
    wid = lax.axis_index("s") * _NC + lax.axis_index("c")
    rows_per_w = n_rows // _NW
    lane = lax.iota(jnp.int32, _LANES)
    lane257 = lane * 257
    ones = jnp.full((_LANES,), 1, jnp.int32)
    kk = jnp.int32(k)

    def key_of(v):
        b = lax.bitcast_convert_type(v, jnp.int32)
        return jnp.where(b < 0, b ^ jnp.int32(0x7FFFFFFF), b)

    def do_row(r, _):
        row = wid * rows_per_w + r
        pltpu.sync_copy(x_hbm.at[row], row_v)

        prefix = jnp.int32(0)
        gt_acc = jnp.int32(0)
        eq_cnt = jnp.int32(0)

        for p in range(4):
            shift = 24 - 8 * p

            def zero_step(j, _c):
                for u in range(_UNROLL):
                    hist_v[pl.ds((j * _UNROLL + u) * _LANES, _LANES)] = (
                        jnp.zeros((_LANES,), jnp.int32)
                    )
                return _c

            lax.fori_loop(0, 264 // _UNROLL, zero_step, jnp.int32(0))

            if p == 0:
                def hist_step(i, _c):
                    for u in range(_UNROLL):
                        s = key_of(
                            row_v[pl.ds((i * _UNROLL + u) * _LANES,
                                        _LANES)]
                        )
                        binv = (s >> 24) + 128
                        plsc.addupdate_scatter(
                            hist_v, [lane257 + binv], ones
                        )
                    return _c
            else:
                himask = jnp.int32(-(1 << (shift + 8)))
                pref_hi = prefix & himask

                def hist_step(i, _c, _himask=himask, _pref=pref_hi,
                              _shift=shift):
                    for u in range(_UNROLL):
                        s = key_of(
                            row_v[pl.ds((i * _UNROLL + u) * _LANES,
                                        _LANES)]
                        )
                        sel = (s & _himask) == _pref
                        binv = (s >> _shift) & 255
                        plsc.addupdate_scatter(
                            hist_v, [lane257 + binv], ones, mask=sel
                        )
                    return _c

            lax.fori_loop(0, nvec // _UNROLL, hist_step, jnp.int32(0))

            remaining = kk - gt_acc
            carry = jnp.int32(0)
            found = jnp.int32(0)
            chosen = jnp.int32(0)
            gt_above = jnp.int32(0)
            hb_ch = jnp.int32(0)
            for g in range(15, -1, -1):
                gv = hist_v[pl.ds(0 * 257 + g * _LANES, _LANES)]
                for l in range(1, _LANES):
                    gv = gv + hist_v[pl.ds(l * 257 + g * _LANES, _LANES)]
                rv = lax.rev(gv, (0,))
                cs = plsc.cumsum(rv)
                tot = jnp.sum(gv)
                hit = (found == 0) & ((carry + tot) >= remaining)
                crossed = (carry + cs) >= remaining
                istar = jnp.min(
                    jnp.where(crossed, lane, jnp.int32(_LANES))
                )
                hb = jnp.sum(jnp.where(lane == istar, rv, 0))
                cs_at = jnp.sum(jnp.where(lane == istar, cs, 0))
                bstar = g * _LANES + (_LANES - 1) - istar
                chosen = jnp.where(hit, bstar, chosen)
                gt_above = jnp.where(hit, carry + cs_at - hb, gt_above)
                hb_ch = jnp.where(hit, hb, hb_ch)
                found = jnp.where(hit, jnp.int32(1), found)
                carry = jnp.where(found == 0, carry + tot, carry)

            byte_bits = (chosen ^ 128) if p == 0 else chosen
            prefix = prefix | (byte_bits << shift)
            gt_acc = gt_acc + gt_above
            eq_cnt = hb_ch

        t = prefix
        need = kk - gt_acc

        @pl.when(eq_cnt == need)
        def _fast():
            def flip_step(i, _c):
                for u in range(_UNROLL):
                    sl = pl.ds((i * _UNROLL + u) * _LANES, _LANES)
                    v = row_v[sl]
                    s = key_of(v)
                    row_v[sl] = jnp.where(s >= t, 1.0 - v, v)
                return _c

            lax.fori_loop(0, nvec // _UNROLL, flip_step, jnp.int32(0))

        @pl.when(eq_cnt != need)
        def _tie():
            def flip_step(i, seen):
                v = row_v[pl.ds(i * _LANES, _LANES)]
                s = key_of(v)
                gtm = s > t
                eqm = s == t
                eqi = eqm.astype(jnp.int32)
                csum = plsc.cumsum(eqi)
                flip = gtm | (eqm & ((csum + seen) <= need))
                row_v[pl.ds(i * _LANES, _LANES)] = jnp.where(
                    flip, 1.0 - v, v
                )
                return seen + jnp.sum(eqi)

            lax.fori_loop(0, nvec, flip_step, jnp.int32(0))

        pltpu.sync_copy(row_v, o_hbm.at[row])
        return _

    lax.fori_loop(0, rows_per_w, do_row, jnp.int32(0))


def _sc_call(flat, k):
    B, L = flat.shape
    mesh = plsc.VectorSubcoreMesh(
        core_axis_name="c", subcore_axis_name="s"
    )
    body = functools.partial(_sc_body, k=k, n_rows=B, length=L)
    fn = pl.kernel(
        body,
        mesh=mesh,
        out_type=jax.ShapeDtypeStruct((B, L), jnp.float32),
        scratch_types=[
            pltpu.VMEM((L,), jnp.float32),
            pltpu.VMEM((4224,), jnp.int32),
        ],
        compiler_params=pltpu.CompilerParams(needs_layout_passes=False),
    )
    return fn(flat)


def kernel(Attention_map):
    B, C, H, W = Attention_map.shape
    L = C * H * W
    k = int(np.clip(int(L * _TOPK_FRAC), 1, C))
    idx_bits = max(int(L - 1).bit_length(), 1)
    flat = Attention_map.reshape(B, L)

    n_sc = _N_SC_ROWS
    rows = _ROWS_PER_BLOCK
    n_tc = B - n_sc
    out_tc = pl.pallas_call(
        functools.partial(
            _flip_topk_kernel, k=k, rows=rows, length=L, idx_bits=idx_bits
        ),
        grid=(n_tc // rows,),
        in_specs=[pl.BlockSpec((rows, L), lambda i: (i, 0))],
        out_specs=pl.BlockSpec((rows, L), lambda i: (i, 0)),
        out_shape=jax.ShapeDtypeStruct((n_tc, L), jnp.float32),
        scratch_shapes=[pltpu.VMEM((rows, 1), jnp.int32)],
        compiler_params=pltpu.CompilerParams(
            dimension_semantics=("parallel",),
        ),
    )(flat[n_sc:])

    out_sc = _sc_call(flat[:n_sc], k)
    out = jnp.concatenate([out_sc, out_tc], axis=0)
    return out.reshape(B, C, H, W)
